# baseline (speedup 1.0000x reference)
import jax
import jax.numpy as jnp
from jax.experimental import pallas as pl
from jax.experimental.pallas import tpu as pltpu
from jax.experimental.pallas import tpu_sc as plsc

D_MODEL = 64
SCALE = 8.0
WINDOW = 128
LANES = 16


def _embed_sc(x_flat, lut):
    num_indices = x_flat.shape[0]
    idx2d = x_flat.reshape(1, num_indices)
    mesh = plsc.VectorSubcoreMesh(core_axis_name="core", subcore_axis_name="subcore")

    @pl.kernel(
        out_type=jax.ShapeDtypeStruct((num_indices, D_MODEL), jnp.float32),
        mesh=mesh,
        compiler_params=pltpu.CompilerParams(use_tc_tiling_on_sc=False),
    )
    def kern(lut_hbm, i_hbm, o_hbm):
        def body(i_vmem, o_vmem):
            pltpu.sync_copy(lut_hbm.at[i_vmem.at[0]], o_vmem)

            @pl.loop(0, WINDOW)
            def _(r):
                for c in range(0, D_MODEL, LANES):
                    slc = (pl.ds(r, 1), pl.ds(c, LANES))
                    o_vmem.at[*slc][...] = o_vmem.at[*slc][...] * SCALE

        pltpu.emit_pipeline(
            body,
            grid=(num_indices // WINDOW,),
            in_specs=[pl.BlockSpec((1, WINDOW), index_map=lambda i: (0, i))],
            out_specs=[pl.BlockSpec((WINDOW, D_MODEL), index_map=lambda i: (i, 0))],
            core_axis_name=("core", "subcore"),
            dimension_semantics=(pltpu.PARALLEL,),
        )(i_hbm, o_hbm)

    return kern(lut, idx2d)


def kernel(x, lut):
    b, s = x.shape
    out = _embed_sc(x.reshape(b * s), lut)
    return out.reshape(b, s, D_MODEL)


# speedup vs baseline: 1.4860x; 1.4860x over previous
import jax
import jax.numpy as jnp
from jax.experimental import pallas as pl
from jax.experimental.pallas import tpu as pltpu
from jax.experimental.pallas import tpu_sc as plsc

D_MODEL = 64
SCALE = 8.0
CHUNK = 128
NBUF = 8
NW = 32
LANES = 16


def _embed_sc(x_flat, lut):
    n = x_flat.shape[0]
    assert n % (NW * CHUNK) == 0
    b_per_w = n // NW
    nchunks = b_per_w // CHUNK
    assert (nchunks - NBUF) % NBUF == 0
    mesh = plsc.VectorSubcoreMesh(core_axis_name="core", subcore_axis_name="subcore")

    @pl.kernel(
        out_type=jax.ShapeDtypeStruct((n, D_MODEL), jnp.float32),
        mesh=mesh,
        compiler_params=pltpu.CompilerParams(use_tc_tiling_on_sc=False),
        scratch_types=[
            pltpu.VMEM((b_per_w,), jnp.int32),
            pltpu.VMEM((NBUF, CHUNK, D_MODEL), jnp.float32),
            pltpu.SemaphoreType.DMA((NBUF,)),
            pltpu.SemaphoreType.DMA((NBUF,)),
            pltpu.SemaphoreType.DMA,
        ],
    )
    def kern(lut_hbm, i_hbm, o_hbm, idx_v, rows_v, gsem, wsem, isem):
        wid = jax.lax.axis_index("subcore") * 2 + jax.lax.axis_index("core")
        base = wid * b_per_w
        pltpu.make_async_copy(i_hbm.at[pl.ds(base, b_per_w)], idx_v, isem).start()
        pltpu.make_async_copy(i_hbm.at[pl.ds(base, b_per_w)], idx_v, isem).wait()

        def gdesc(c, b):
            return pltpu.make_async_copy(
                lut_hbm.at[idx_v.at[pl.ds(c * CHUNK, CHUNK)]],
                rows_v.at[b],
                gsem.at[b],
            )

        def wdesc(c, b):
            return pltpu.make_async_copy(
                rows_v.at[b],
                o_hbm.at[pl.ds(base + c * CHUNK, CHUNK)],
                wsem.at[b],
            )

        def scale(b):
            buf = rows_v.at[b]

            @pl.loop(0, CHUNK, step=4)
            def _(r):
                for dr in range(4):
                    for c in range(0, D_MODEL, LANES):
                        slc = (pl.ds(r + dr, 1), pl.ds(c, LANES))
                        buf.at[*slc][...] = buf.at[*slc][...] * SCALE

        for b in range(NBUF):
            gdesc(b, b).start()

        @pl.loop(0, nchunks - NBUF, step=NBUF)
        def _(g):
            for b in range(NBUF):
                cur = g + b
                gdesc(cur, b).wait()
                scale(b)
                wdesc(cur, b).start()
            for b in range(NBUF):
                cur = g + b
                wdesc(cur, b).wait()
                gdesc(cur + NBUF, b).start()

        for b in range(NBUF):
            cur = nchunks - NBUF + b
            gdesc(cur, b).wait()
            scale(b)
            wdesc(cur, b).start()
        for b in range(NBUF):
            cur = nchunks - NBUF + b
            wdesc(cur, b).wait()

    return kern(lut, x_flat)


def kernel(x, lut):
    b, s = x.shape
    out = _embed_sc(x.reshape(b * s), lut)
    return out.reshape(b, s, D_MODEL)
